# baseline (speedup 1.0000x reference)
import functools

import jax
import jax.numpy as jnp
from jax import lax
from jax.experimental import pallas as pl
from jax.experimental.pallas import tpu as pltpu
from jax.experimental.pallas import tpu_sc as plsc

_EPS = 1e-5

_NC = 2
_NS = 16
_NW = _NC * _NS
_G = 128
_CC = 128


def _pad_dst(n):
    q = _NS * 8
    return ((n + q - 1) // q) * q


def _chunks(total, step):
    out = []
    o = 0
    while o < total:
        s = min(step, total - o)
        out.append((o, s))
        o += s
    return out


def _seg_sum_body(n_pad, nh, h, x_hbm, src_hbm, dst_hbm, out_hbm,
                  src_v, dst_v, rows0_v, rows1_v, acc_sh, sem0, sem1):
    cid = lax.axis_index("c")
    sid = lax.axis_index("s")
    wid = cid * _NS + sid
    rows_per_sub = n_pad // _NS

    @pl.loop(0, _G)
    def _(i):
        for c in range(h // 16):
            rows1_v[i, pl.ds(c * 16, 16)] = jnp.zeros((16,), jnp.float32)

    for off, sz in _chunks(rows_per_sub, _G):
        pltpu.sync_copy(rows1_v.at[pl.ds(0, sz)],
                        acc_sh.at[pl.ds(sid * rows_per_sub + off, sz)])

    plsc.subcore_barrier()

    for half in range(2):
        pltpu.sync_copy(src_hbm.at[wid, half], src_v)
        pltpu.sync_copy(dst_hbm.at[wid, half], dst_v)
        pltpu.async_copy(x_hbm.at[src_v.at[0]], rows0_v, sem0)

        @pl.loop(0, nh // 2)
        def _(k):
            j0 = 2 * k
            j1 = 2 * k + 1
            pltpu.make_async_copy(x_hbm.at[src_v.at[j0]], rows0_v, sem0).wait()
            pltpu.async_copy(x_hbm.at[src_v.at[j1]], rows1_v, sem1)
            pltpu.sync_copy(rows0_v, acc_sh.at[dst_v.at[j0]], add=True)
            pltpu.make_async_copy(x_hbm.at[src_v.at[j1]], rows1_v, sem1).wait()

            @pl.when(j1 + 1 < nh)
            def _():
                pltpu.async_copy(x_hbm.at[src_v.at[j1 + 1]], rows0_v, sem0)

            pltpu.sync_copy(rows1_v, acc_sh.at[dst_v.at[j1]], add=True)

    plsc.subcore_barrier()

    for off, sz in _chunks(rows_per_sub, 128):
        base = sid * rows_per_sub + off
        pltpu.sync_copy(acc_sh.at[pl.ds(base, sz)],
                        out_hbm.at[cid].at[pl.ds(base, sz)])


def _make_seg_sum(n_pad, nh, h):
    mesh = plsc.VectorSubcoreMesh(core_axis_name="c", subcore_axis_name="s")
    assert nh % 2 == 0
    return pl.kernel(
        functools.partial(_seg_sum_body, n_pad, nh, h),
        out_type=jax.ShapeDtypeStruct((_NC, n_pad, h), jnp.float32),
        mesh=mesh,
        scratch_types=[
            pltpu.VMEM((nh, _G), jnp.int32),
            pltpu.VMEM((nh, _G), jnp.int32),
            pltpu.VMEM((_G, h), jnp.float32),
            pltpu.VMEM((_G, h), jnp.float32),
            pltpu.VMEM_SHARED((n_pad, h), jnp.float32),
            pltpu.SemaphoreType.DMA,
            pltpu.SemaphoreType.DMA,
        ],
    )


def _counts_body(n_pad, n_chunk, dui_hbm, diu_hbm, cu_hbm, ci_hbm,
                 dst_v, ones_v, acc_sh):
    cid = lax.axis_index("c")
    sid = lax.axis_index("s")
    rows_per_sub = n_pad // _NS

    @pl.loop(0, _CC)
    def _(i):
        for c in range(8):
            ones_v[i, pl.ds(c * 16, 16)] = jnp.zeros((16,), jnp.float32)

    for off, sz in _chunks(rows_per_sub, _CC):
        pltpu.sync_copy(ones_v.at[pl.ds(0, sz)],
                        acc_sh.at[pl.ds(sid * rows_per_sub + off, sz)])

    @pl.loop(0, _CC)
    def _(i):
        for c in range(8):
            ones_v[i, pl.ds(c * 16, 16)] = jnp.ones((16,), jnp.float32)

    @pl.when(cid == 0)
    def _():
        pltpu.sync_copy(diu_hbm.at[sid], dst_v)

    @pl.when(cid == 1)
    def _():
        pltpu.sync_copy(dui_hbm.at[sid], dst_v)

    plsc.subcore_barrier()

    @pl.loop(0, n_chunk)
    def _(j):
        pltpu.sync_copy(ones_v, acc_sh.at[dst_v.at[j]], add=True)

    plsc.subcore_barrier()

    for off, sz in _chunks(rows_per_sub, 128):
        base = sid * rows_per_sub + off

        @pl.when(cid == 0)
        def _():
            pltpu.sync_copy(acc_sh.at[pl.ds(base, sz)],
                            cu_hbm.at[pl.ds(base, sz)])

        @pl.when(cid == 1)
        def _():
            pltpu.sync_copy(acc_sh.at[pl.ds(base, sz)],
                            ci_hbm.at[pl.ds(base, sz)])


def _make_counts(n_pad, n_chunk):
    mesh = plsc.VectorSubcoreMesh(core_axis_name="c", subcore_axis_name="s")
    out = jax.ShapeDtypeStruct((n_pad, 128), jnp.float32)
    return pl.kernel(
        functools.partial(_counts_body, n_pad, n_chunk),
        out_type=[out, out],
        mesh=mesh,
        scratch_types=[
            pltpu.VMEM((n_chunk, _CC), jnp.int32),
            pltpu.VMEM((_CC, 128), jnp.float32),
            pltpu.VMEM_SHARED((n_pad, 128), jnp.float32),
        ],
    )


def _bn_relu(z, g, b):
    mu = jnp.mean(z, axis=0, keepdims=True)
    var = jnp.mean((z - mu) ** 2, axis=0, keepdims=True)
    return jnp.maximum((z - mu) / jnp.sqrt(var + _EPS) * g + b, 0.0)


def _prep_body(xu, xi, Wu, bu, gu, bbu, Wi, bi, gi, bbi, hu_o, hi_o):
    zu = jnp.dot(xu[...], Wu[...], preferred_element_type=jnp.float32) + bu[...]
    hu_o[...] = _bn_relu(zu, gu[...], bbu[...])
    zi = jnp.dot(xi[...], Wi[...], preferred_element_type=jnp.float32) + bi[...]
    hi_o[...] = _bn_relu(zi, gi[...], bbi[...])


def _layer_body(P, cnt, h, Wl, bl, Wr, g, bb, out):
    n = h.shape[0]
    Pf = P[...]
    s = Pf[0, :n] + Pf[1, :n]
    c = cnt[...][:n, :1]
    agg = s / jnp.maximum(c, 1.0)
    z = (jnp.dot(agg, Wl[...], preferred_element_type=jnp.float32)
         + bl[...]
         + jnp.dot(h[...], Wr[...], preferred_element_type=jnp.float32))
    out[...] = _bn_relu(z, g[...], bb[...]) + h[...]


def _final_body(hu, hi, W, b, ou, oi):
    ou[...] = jnp.dot(hu[...], W[...], preferred_element_type=jnp.float32) + b[...]
    oi[...] = jnp.dot(hi[...], W[...], preferred_element_type=jnp.float32) + b[...]


def _r2(v):
    return v.reshape(1, -1)


def kernel(x_user, x_item, edge_ui, edge_iu, params):
    p = params
    N_U, H = x_user.shape
    N_I = x_item.shape[0]
    E = edge_ui.shape[1]
    f32 = jnp.float32
    n_pad = _pad_dst(max(N_U, N_I))
    trash = n_pad - 1

    q = 4 * _G * _NW
    e_seg = ((E + q - 1) // q) * q
    per_w = e_seg // _NW
    nh = per_w // _G // 2

    def padded(row, fill):
        return jnp.concatenate(
            [row, jnp.full((e_seg - E,), fill, jnp.int32)])

    src_ui = padded(edge_ui[0], 0).reshape(_NW, 2, nh, _G)
    dst_ui = padded(edge_ui[1], trash).reshape(_NW, 2, nh, _G)
    src_iu = padded(edge_iu[0], 0).reshape(_NW, 2, nh, _G)
    dst_iu = padded(edge_iu[1], trash).reshape(_NW, 2, nh, _G)

    qc = _CC * _NS
    e_cnt = ((E + qc - 1) // qc) * qc
    nc_cnt = e_cnt // _NS // _CC

    def padded_c(row):
        return jnp.concatenate(
            [row, jnp.full((e_cnt - E,), trash, jnp.int32)])

    dst_ui_c = padded_c(edge_ui[1]).reshape(_NS, nc_cnt, _CC)
    dst_iu_c = padded_c(edge_iu[1]).reshape(_NS, nc_cnt, _CC)

    prep = pl.pallas_call(
        _prep_body,
        out_shape=[jax.ShapeDtypeStruct((N_U, H), f32),
                   jax.ShapeDtypeStruct((N_I, H), f32)],
    )
    h_u, h_i = prep(x_user, x_item,
                    p['lin_user_W'], _r2(p['lin_user_b']),
                    _r2(p['in_bn_user_g']), _r2(p['in_bn_user_b']),
                    p['lin_item_W'], _r2(p['lin_item_b']),
                    _r2(p['in_bn_item_g']), _r2(p['in_bn_item_b']))

    cnt_u, cnt_i = _make_counts(n_pad, nc_cnt)(dst_ui_c, dst_iu_c)

    seg = _make_seg_sum(n_pad, nh, H)

    layer = pl.pallas_call(
        _layer_body,
        out_shape=jax.ShapeDtypeStruct((N_U, H), f32),
    )
    for l in range(3):
        P_i = seg(h_u, src_ui, dst_ui)
        P_u = seg(h_i, src_iu, dst_iu)
        h_i_new = layer(P_i, cnt_i, h_i,
                        p[f'c{l}_ui_Wl'], _r2(p[f'c{l}_ui_bl']),
                        p[f'c{l}_ui_Wr'], _r2(p[f'bn{l}_i_g']), _r2(p[f'bn{l}_i_b']))
        h_u_new = layer(P_u, cnt_u, h_u,
                        p[f'c{l}_iu_Wl'], _r2(p[f'c{l}_iu_bl']),
                        p[f'c{l}_iu_Wr'], _r2(p[f'bn{l}_u_g']), _r2(p[f'bn{l}_u_b']))
        h_u, h_i = h_u_new, h_i_new

    final = pl.pallas_call(
        _final_body,
        out_shape=[jax.ShapeDtypeStruct((N_U, p['final_W'].shape[1]), f32),
                   jax.ShapeDtypeStruct((N_I, p['final_W'].shape[1]), f32)],
    )
    return final(h_u, h_i, p['final_W'], _r2(p['final_b']))


# speedup vs baseline: 1.4752x; 1.4752x over previous
import functools

import jax
import jax.numpy as jnp
from jax import lax
from jax.experimental import pallas as pl
from jax.experimental.pallas import tpu as pltpu
from jax.experimental.pallas import tpu_sc as plsc

_EPS = 1e-5

_NC = 2
_NS = 16
_G = 128
_CC = 128


def _pad_dst(n):
    q = _NS * 8
    return ((n + q - 1) // q) * q


def _chunks(total, step):
    out = []
    o = 0
    while o < total:
        s = min(step, total - o)
        out.append((o, s))
        o += s
    return out


def _seg_sum_body(n_pad, nh, h, x_hbm, src_hbm, dst_hbm, out_hbm,
                  src_v, dst_v, rows_v, acc_sh, sem):
    cid = lax.axis_index("c")
    sid = lax.axis_index("s")
    wid = cid * _NS + sid
    rows_per_sub = n_pad // _NS

    @pl.loop(0, _G)
    def _(i):
        for c in range(h // 16):
            rows_v[i, pl.ds(c * 16, 16)] = jnp.zeros((16,), jnp.float32)

    for off, sz in _chunks(rows_per_sub, _G):
        pltpu.sync_copy(rows_v.at[pl.ds(0, sz)],
                        acc_sh.at[pl.ds(sid * rows_per_sub + off, sz)])

    plsc.subcore_barrier()

    for half in range(2):
        pltpu.sync_copy(src_hbm.at[wid, half], src_v)
        pltpu.sync_copy(dst_hbm.at[wid, half], dst_v)

        @pl.loop(0, nh)
        def _(j):
            pltpu.async_copy(x_hbm.at[src_v.at[j]], rows_v, sem).wait()
            pltpu.sync_copy(rows_v, acc_sh.at[dst_v.at[j]], add=True)

    plsc.subcore_barrier()

    for off, sz in _chunks(rows_per_sub, 128):
        base = sid * rows_per_sub + off
        pltpu.sync_copy(acc_sh.at[pl.ds(base, sz)],
                        out_hbm.at[cid].at[pl.ds(base, sz)])


def _make_seg_sum(n_pad, nh, h):
    mesh = plsc.VectorSubcoreMesh(core_axis_name="c", subcore_axis_name="s")
    return pl.kernel(
        functools.partial(_seg_sum_body, n_pad, nh, h),
        out_type=jax.ShapeDtypeStruct((_NC, n_pad, h), jnp.float32),
        mesh=mesh,
        scratch_types=[
            pltpu.VMEM((nh, _G), jnp.int32),
            pltpu.VMEM((nh, _G), jnp.int32),
            pltpu.VMEM((_G, h), jnp.float32),
            pltpu.VMEM_SHARED((n_pad, h), jnp.float32),
            pltpu.SemaphoreType.DMA,
        ],
    )


def _counts_body(n_pad, n_chunk, dst_hbm, cnt_hbm, dst_v, ones_v, acc_sh):
    cid = lax.axis_index("c")
    sid = lax.axis_index("s")
    wid = cid * _NS + sid
    rows_per_sub = n_pad // _NS

    @pl.loop(0, _CC)
    def _(i):
        for c in range(8):
            ones_v[i, pl.ds(c * 16, 16)] = jnp.zeros((16,), jnp.float32)

    for off, sz in _chunks(rows_per_sub, _CC):
        pltpu.sync_copy(ones_v.at[pl.ds(0, sz)],
                        acc_sh.at[pl.ds(sid * rows_per_sub + off, sz)])

    @pl.loop(0, _CC)
    def _(i):
        for c in range(8):
            ones_v[i, pl.ds(c * 16, 16)] = jnp.ones((16,), jnp.float32)

    pltpu.sync_copy(dst_hbm.at[wid], dst_v)
    plsc.subcore_barrier()

    @pl.loop(0, n_chunk)
    def _(j):
        pltpu.sync_copy(ones_v, acc_sh.at[dst_v.at[j]], add=True)

    plsc.subcore_barrier()

    for off, sz in _chunks(rows_per_sub, 128):
        base = sid * rows_per_sub + off
        pltpu.sync_copy(acc_sh.at[pl.ds(base, sz)],
                        cnt_hbm.at[cid].at[pl.ds(base, sz)])


def _make_counts(n_pad, n_chunk):
    mesh = plsc.VectorSubcoreMesh(core_axis_name="c", subcore_axis_name="s")
    return pl.kernel(
        functools.partial(_counts_body, n_pad, n_chunk),
        out_type=jax.ShapeDtypeStruct((_NC, n_pad, 128), jnp.float32),
        mesh=mesh,
        scratch_types=[
            pltpu.VMEM((n_chunk, _CC), jnp.int32),
            pltpu.VMEM((_CC, 128), jnp.float32),
            pltpu.VMEM_SHARED((n_pad, 128), jnp.float32),
        ],
    )


def _bn_relu(z, g, b):
    mu = jnp.mean(z, axis=0, keepdims=True)
    var = jnp.mean((z - mu) ** 2, axis=0, keepdims=True)
    return jnp.maximum((z - mu) / jnp.sqrt(var + _EPS) * g + b, 0.0)


def _prep_body(xu, xi, Wu, bu, gu, bbu, Wi, bi, gi, bbi, h_o):
    zi = jnp.dot(xi[...], Wi[...], preferred_element_type=jnp.float32) + bi[...]
    h_o[0] = _bn_relu(zi, gi[...], bbi[...])
    zu = jnp.dot(xu[...], Wu[...], preferred_element_type=jnp.float32) + bu[...]
    h_o[1] = _bn_relu(zu, gu[...], bbu[...])


def _layer_body(P, cnt, h,
                Wl_ui, bl_ui, Wr_ui, g_i, bb_i,
                Wl_iu, bl_iu, Wr_iu, g_u, bb_u, out):
    n = h.shape[1]
    h_i = h[0]
    h_u = h[1]

    agg_i = P[1, :n] / jnp.maximum(cnt[1, :n, :1], 1.0)
    z_i = (jnp.dot(agg_i, Wl_ui[...], preferred_element_type=jnp.float32)
           + bl_ui[...]
           + jnp.dot(h_i, Wr_ui[...], preferred_element_type=jnp.float32))
    out[0] = _bn_relu(z_i, g_i[...], bb_i[...]) + h_i

    agg_u = P[0, :n] / jnp.maximum(cnt[0, :n, :1], 1.0)
    z_u = (jnp.dot(agg_u, Wl_iu[...], preferred_element_type=jnp.float32)
           + bl_iu[...]
           + jnp.dot(h_u, Wr_iu[...], preferred_element_type=jnp.float32))
    out[1] = _bn_relu(z_u, g_u[...], bb_u[...]) + h_u


def _final_body(h, W, b, ou, oi):
    ou[...] = jnp.dot(h[1], W[...], preferred_element_type=jnp.float32) + b[...]
    oi[...] = jnp.dot(h[0], W[...], preferred_element_type=jnp.float32) + b[...]


def _r2(v):
    return v.reshape(1, -1)


def kernel(x_user, x_item, edge_ui, edge_iu, params):
    p = params
    N_U, H = x_user.shape
    N_I = x_item.shape[0]
    assert N_U == N_I
    E = edge_ui.shape[1]
    f32 = jnp.float32
    n_pad = _pad_dst(max(N_U, N_I))
    trash = n_pad - 1

    q = 2 * _G * _NS
    e_seg = ((E + q - 1) // q) * q
    per_s = e_seg // _NS
    nh = per_s // _G // 2

    def padded(row, fill):
        return jnp.concatenate(
            [row, jnp.full((e_seg - E,), fill, jnp.int32)])

    src_all = jnp.concatenate([
        padded(edge_iu[0], 0).reshape(_NS, 2, nh, _G),
        padded(edge_ui[0] + N_U, 0).reshape(_NS, 2, nh, _G)])
    dst_all = jnp.concatenate([
        padded(edge_iu[1], trash).reshape(_NS, 2, nh, _G),
        padded(edge_ui[1], trash).reshape(_NS, 2, nh, _G)])

    qc = _CC * _NS
    e_cnt = ((E + qc - 1) // qc) * qc
    nc_cnt = e_cnt // _NS // _CC

    def padded_c(row):
        return jnp.concatenate(
            [row, jnp.full((e_cnt - E,), trash, jnp.int32)])

    dst_cnt = jnp.concatenate([
        padded_c(edge_iu[1]).reshape(_NS, nc_cnt, _CC),
        padded_c(edge_ui[1]).reshape(_NS, nc_cnt, _CC)])

    prep = pl.pallas_call(
        _prep_body,
        out_shape=jax.ShapeDtypeStruct((2, N_U, H), f32),
    )
    h = prep(x_user, x_item,
             p['lin_user_W'], _r2(p['lin_user_b']),
             _r2(p['in_bn_user_g']), _r2(p['in_bn_user_b']),
             p['lin_item_W'], _r2(p['lin_item_b']),
             _r2(p['in_bn_item_g']), _r2(p['in_bn_item_b']))

    cnt = _make_counts(n_pad, nc_cnt)(dst_cnt)

    seg = _make_seg_sum(n_pad, nh, H)

    layer = pl.pallas_call(
        _layer_body,
        out_shape=jax.ShapeDtypeStruct((2, N_U, H), f32),
    )
    for l in range(3):
        P = seg(h.reshape(2 * N_U, H), src_all, dst_all)
        h = layer(P, cnt, h,
                  p[f'c{l}_ui_Wl'], _r2(p[f'c{l}_ui_bl']), p[f'c{l}_ui_Wr'],
                  _r2(p[f'bn{l}_i_g']), _r2(p[f'bn{l}_i_b']),
                  p[f'c{l}_iu_Wl'], _r2(p[f'c{l}_iu_bl']), p[f'c{l}_iu_Wr'],
                  _r2(p[f'bn{l}_u_g']), _r2(p[f'bn{l}_u_b']))

    final = pl.pallas_call(
        _final_body,
        out_shape=[jax.ShapeDtypeStruct((N_U, p['final_W'].shape[1]), f32),
                   jax.ShapeDtypeStruct((N_I, p['final_W'].shape[1]), f32)],
    )
    return final(h, p['final_W'], _r2(p['final_b']))


# speedup vs baseline: 1.4810x; 1.0039x over previous
import functools

import jax
import jax.numpy as jnp
from jax import lax
from jax.experimental import pallas as pl
from jax.experimental.pallas import tpu as pltpu
from jax.experimental.pallas import tpu_sc as plsc

_EPS = 1e-5

_NC = 2
_NS = 16
_G = 128
_CC = 128


def _pad_dst(n):
    q = _NS * 8
    return ((n + q - 1) // q) * q


def _chunks(total, step):
    out = []
    o = 0
    while o < total:
        s = min(step, total - o)
        out.append((o, s))
        o += s
    return out


def _seg_sum_body(n_pad, nh, h, x_hbm, src_hbm, dst_hbm, out_hbm,
                  src_v, dst_v, rows_v, acc_sh, sem):
    cid = lax.axis_index("c")
    sid = lax.axis_index("s")
    wid = cid * _NS + sid
    rows_per_sub = n_pad // _NS

    @pl.loop(0, _G)
    def _(i):
        for c in range(h // 16):
            rows_v[i, pl.ds(c * 16, 16)] = jnp.zeros((16,), jnp.float32)

    for off, sz in _chunks(rows_per_sub, _G):
        pltpu.sync_copy(rows_v.at[pl.ds(0, sz)],
                        acc_sh.at[pl.ds(sid * rows_per_sub + off, sz)])

    plsc.subcore_barrier()

    for half in range(2):
        pltpu.sync_copy(src_hbm.at[wid, half], src_v)
        pltpu.sync_copy(dst_hbm.at[wid, half], dst_v)

        @pl.loop(0, nh)
        def _(j):
            pltpu.async_copy(x_hbm.at[src_v.at[j]], rows_v, sem).wait()
            pltpu.sync_copy(rows_v, acc_sh.at[dst_v.at[j]], add=True)

    plsc.subcore_barrier()

    for off, sz in _chunks(rows_per_sub, 128):
        base = sid * rows_per_sub + off
        pltpu.sync_copy(acc_sh.at[pl.ds(base, sz)],
                        out_hbm.at[cid].at[pl.ds(base, sz)])


def _make_seg_sum(n_pad, nh, h):
    mesh = plsc.VectorSubcoreMesh(core_axis_name="c", subcore_axis_name="s")
    return pl.kernel(
        functools.partial(_seg_sum_body, n_pad, nh, h),
        out_type=jax.ShapeDtypeStruct((_NC, n_pad, h), jnp.float32),
        mesh=mesh,
        scratch_types=[
            pltpu.VMEM((nh, _G), jnp.int32),
            pltpu.VMEM((nh, _G), jnp.int32),
            pltpu.VMEM((_G, h), jnp.float32),
            pltpu.VMEM_SHARED((n_pad, h), jnp.float32),
            pltpu.SemaphoreType.DMA,
        ],
    )


def _counts_body(n_pad, n_chunk, dst_hbm, cnt_hbm, dst_v, ones_v, acc_sh):
    cid = lax.axis_index("c")
    sid = lax.axis_index("s")
    wid = cid * _NS + sid
    rows_per_sub = n_pad // _NS

    @pl.loop(0, _CC)
    def _(i):
        for c in range(8):
            ones_v[i, pl.ds(c * 16, 16)] = jnp.zeros((16,), jnp.float32)

    for off, sz in _chunks(rows_per_sub, _CC):
        pltpu.sync_copy(ones_v.at[pl.ds(0, sz)],
                        acc_sh.at[pl.ds(sid * rows_per_sub + off, sz)])

    @pl.loop(0, _CC)
    def _(i):
        for c in range(8):
            ones_v[i, pl.ds(c * 16, 16)] = jnp.ones((16,), jnp.float32)

    pltpu.sync_copy(dst_hbm.at[wid], dst_v)
    plsc.subcore_barrier()

    @pl.loop(0, n_chunk)
    def _(j):
        pltpu.sync_copy(ones_v, acc_sh.at[dst_v.at[j]], add=True)

    plsc.subcore_barrier()

    for off, sz in _chunks(rows_per_sub, 128):
        base = sid * rows_per_sub + off
        pltpu.sync_copy(acc_sh.at[pl.ds(base, sz)],
                        cnt_hbm.at[cid].at[pl.ds(base, sz)])


def _make_counts(n_pad, n_chunk):
    mesh = plsc.VectorSubcoreMesh(core_axis_name="c", subcore_axis_name="s")
    return pl.kernel(
        functools.partial(_counts_body, n_pad, n_chunk),
        out_type=jax.ShapeDtypeStruct((_NC, n_pad, 128), jnp.float32),
        mesh=mesh,
        scratch_types=[
            pltpu.VMEM((n_chunk, _CC), jnp.int32),
            pltpu.VMEM((_CC, 128), jnp.float32),
            pltpu.VMEM_SHARED((n_pad, 128), jnp.float32),
        ],
    )


def _bn_relu(z, g, b):
    mu = jnp.mean(z, axis=0, keepdims=True)
    var = jnp.mean((z - mu) ** 2, axis=0, keepdims=True)
    return jnp.maximum((z - mu) / jnp.sqrt(var + _EPS) * g + b, 0.0)


def _prep_body(xu, xi, Wu, bu, gu, bbu, Wi, bi, gi, bbi, h_o):
    zi = jnp.dot(xi[...], Wi[...], preferred_element_type=jnp.float32) + bi[...]
    h_o[0] = _bn_relu(zi, gi[...], bbi[...])
    zu = jnp.dot(xu[...], Wu[...], preferred_element_type=jnp.float32) + bu[...]
    h_o[1] = _bn_relu(zu, gu[...], bbu[...])


def _layer_body(P, cnt, h,
                Wl_ui, bl_ui, Wr_ui, g_i, bb_i,
                Wl_iu, bl_iu, Wr_iu, g_u, bb_u, out):
    n = h.shape[1]
    h_i = h[0]
    h_u = h[1]

    agg_i = P[1, :n] / jnp.maximum(cnt[1, :n, :1], 1.0)
    z_i = (jnp.dot(agg_i, Wl_ui[...], preferred_element_type=jnp.float32)
           + bl_ui[...]
           + jnp.dot(h_i, Wr_ui[...], preferred_element_type=jnp.float32))
    out[0] = _bn_relu(z_i, g_i[...], bb_i[...]) + h_i

    agg_u = P[0, :n] / jnp.maximum(cnt[0, :n, :1], 1.0)
    z_u = (jnp.dot(agg_u, Wl_iu[...], preferred_element_type=jnp.float32)
           + bl_iu[...]
           + jnp.dot(h_u, Wr_iu[...], preferred_element_type=jnp.float32))
    out[1] = _bn_relu(z_u, g_u[...], bb_u[...]) + h_u


def _final_body(h, W, b, ou, oi):
    ou[...] = jnp.dot(h[1], W[...], preferred_element_type=jnp.float32) + b[...]
    oi[...] = jnp.dot(h[0], W[...], preferred_element_type=jnp.float32) + b[...]


def _r2(v):
    return v.reshape(1, -1)


def kernel(x_user, x_item, edge_ui, edge_iu, params):
    p = params
    N_U, H = x_user.shape
    N_I = x_item.shape[0]
    assert N_U == N_I
    E = edge_ui.shape[1]
    f32 = jnp.float32
    n_pad = _pad_dst(max(N_U, N_I))
    n_trash = n_pad - max(N_U, N_I)

    def trash_rows(k):
        return max(N_U, N_I) + (jnp.arange(k, dtype=jnp.int32) % n_trash)

    q = 2 * _G * _NS
    e_seg = ((E + q - 1) // q) * q
    per_s = e_seg // _NS
    nh = per_s // _G // 2

    def padded(row, fill):
        if fill is None:
            tail = jnp.zeros((e_seg - E,), jnp.int32)
        else:
            tail = fill
        return jnp.concatenate([row, tail])

    tr_seg = trash_rows(e_seg - E)
    src_all = jnp.concatenate([
        padded(edge_iu[0], None).reshape(_NS, 2, nh, _G),
        padded(edge_ui[0] + N_U, None).reshape(_NS, 2, nh, _G)])
    dst_all = jnp.concatenate([
        padded(edge_iu[1], tr_seg).reshape(_NS, 2, nh, _G),
        padded(edge_ui[1], tr_seg).reshape(_NS, 2, nh, _G)])

    qc = _CC * _NS
    e_cnt = ((E + qc - 1) // qc) * qc
    nc_cnt = e_cnt // _NS // _CC

    def padded_c(row):
        return jnp.concatenate([row, trash_rows(e_cnt - E)])

    dst_cnt = jnp.concatenate([
        padded_c(edge_iu[1]).reshape(_NS, nc_cnt, _CC),
        padded_c(edge_ui[1]).reshape(_NS, nc_cnt, _CC)])

    prep = pl.pallas_call(
        _prep_body,
        out_shape=jax.ShapeDtypeStruct((2, N_U, H), f32),
    )
    h = prep(x_user, x_item,
             p['lin_user_W'], _r2(p['lin_user_b']),
             _r2(p['in_bn_user_g']), _r2(p['in_bn_user_b']),
             p['lin_item_W'], _r2(p['lin_item_b']),
             _r2(p['in_bn_item_g']), _r2(p['in_bn_item_b']))

    cnt = _make_counts(n_pad, nc_cnt)(dst_cnt)

    seg = _make_seg_sum(n_pad, nh, H)

    layer = pl.pallas_call(
        _layer_body,
        out_shape=jax.ShapeDtypeStruct((2, N_U, H), f32),
    )
    for l in range(3):
        P = seg(h.reshape(2 * N_U, H), src_all, dst_all)
        h = layer(P, cnt, h,
                  p[f'c{l}_ui_Wl'], _r2(p[f'c{l}_ui_bl']), p[f'c{l}_ui_Wr'],
                  _r2(p[f'bn{l}_i_g']), _r2(p[f'bn{l}_i_b']),
                  p[f'c{l}_iu_Wl'], _r2(p[f'c{l}_iu_bl']), p[f'c{l}_iu_Wr'],
                  _r2(p[f'bn{l}_u_g']), _r2(p[f'bn{l}_u_b']))

    final = pl.pallas_call(
        _final_body,
        out_shape=[jax.ShapeDtypeStruct((N_U, p['final_W'].shape[1]), f32),
                   jax.ShapeDtypeStruct((N_I, p['final_W'].shape[1]), f32)],
    )
    return final(h, p['final_W'], _r2(p['final_b']))


# speedup vs baseline: 1.9197x; 1.2962x over previous
import functools

import jax
import jax.numpy as jnp
from jax import lax
from jax.experimental import pallas as pl
from jax.experimental.pallas import tpu as pltpu
from jax.experimental.pallas import tpu_sc as plsc

_EPS = 1e-5

_NC = 2
_NS = 16
_NW = _NC * _NS
_C = 80
_CC = 128


def _pad_dst(n):
    q = _NS * 8
    return ((n + q - 1) // q) * q


def _chunks(total, step):
    out = []
    o = 0
    while o < total:
        s = min(step, total - o)
        out.append((o, s))
        o += s
    return out


def _seg_sum_body(n_pad, n_chunk, h, x_hbm, src_hbm, dst_hbm, out_hbm,
                  src_v, dst_v, rows_v, zero_v, acc_sh, sem):
    cid = lax.axis_index("c")
    sid = lax.axis_index("s")
    wid = cid * _NS + sid
    rows_per_sub = n_pad // _NS

    @pl.loop(0, 8)
    def _(i):
        for c in range(h // 16):
            zero_v[i, pl.ds(c * 16, 16)] = jnp.zeros((16,), jnp.float32)

    for off, sz in _chunks(rows_per_sub, 8):
        pltpu.sync_copy(zero_v.at[pl.ds(0, sz)],
                        acc_sh.at[pl.ds(sid * rows_per_sub + off, sz)])

    pltpu.sync_copy(src_hbm.at[wid], src_v)
    pltpu.sync_copy(dst_hbm.at[wid], dst_v)
    plsc.subcore_barrier()

    @pl.loop(0, n_chunk)
    def _(j):
        pltpu.async_copy(x_hbm.at[src_v.at[j]], rows_v, sem).wait()
        pltpu.sync_copy(rows_v, acc_sh.at[dst_v.at[j]], add=True)

    plsc.subcore_barrier()

    for off, sz in _chunks(rows_per_sub, 128):
        base = sid * rows_per_sub + off
        pltpu.sync_copy(acc_sh.at[pl.ds(base, sz)],
                        out_hbm.at[cid].at[pl.ds(base, sz)])


def _make_seg_sum(n_pad, n_chunk, h):
    mesh = plsc.VectorSubcoreMesh(core_axis_name="c", subcore_axis_name="s")
    return pl.kernel(
        functools.partial(_seg_sum_body, n_pad, n_chunk, h),
        out_type=jax.ShapeDtypeStruct((_NC, n_pad, h), jnp.float32),
        mesh=mesh,
        scratch_types=[
            pltpu.VMEM((n_chunk, _C), jnp.int32),
            pltpu.VMEM((n_chunk, _C), jnp.int32),
            pltpu.VMEM((_C, h), jnp.float32),
            pltpu.VMEM((8, h), jnp.float32),
            pltpu.VMEM_SHARED((n_pad, h), jnp.float32),
            pltpu.SemaphoreType.DMA,
        ],
    )


def _counts_body(n_pad, n_chunk, dst_hbm, cnt_hbm, dst_v, ones_v, acc_sh):
    cid = lax.axis_index("c")
    sid = lax.axis_index("s")
    wid = cid * _NS + sid
    rows_per_sub = n_pad // _NS

    @pl.loop(0, _CC)
    def _(i):
        for c in range(8):
            ones_v[i, pl.ds(c * 16, 16)] = jnp.zeros((16,), jnp.float32)

    for off, sz in _chunks(rows_per_sub, _CC):
        pltpu.sync_copy(ones_v.at[pl.ds(0, sz)],
                        acc_sh.at[pl.ds(sid * rows_per_sub + off, sz)])

    @pl.loop(0, _CC)
    def _(i):
        for c in range(8):
            ones_v[i, pl.ds(c * 16, 16)] = jnp.ones((16,), jnp.float32)

    pltpu.sync_copy(dst_hbm.at[wid], dst_v)
    plsc.subcore_barrier()

    @pl.loop(0, n_chunk)
    def _(j):
        pltpu.sync_copy(ones_v, acc_sh.at[dst_v.at[j]], add=True)

    plsc.subcore_barrier()

    for off, sz in _chunks(rows_per_sub, 128):
        base = sid * rows_per_sub + off
        pltpu.sync_copy(acc_sh.at[pl.ds(base, sz)],
                        cnt_hbm.at[cid].at[pl.ds(base, sz)])


def _make_counts(n_pad, n_chunk):
    mesh = plsc.VectorSubcoreMesh(core_axis_name="c", subcore_axis_name="s")
    return pl.kernel(
        functools.partial(_counts_body, n_pad, n_chunk),
        out_type=jax.ShapeDtypeStruct((_NC, n_pad, 128), jnp.float32),
        mesh=mesh,
        scratch_types=[
            pltpu.VMEM((n_chunk, _CC), jnp.int32),
            pltpu.VMEM((_CC, 128), jnp.float32),
            pltpu.VMEM_SHARED((n_pad, 128), jnp.float32),
        ],
    )


def _bn_relu(z, g, b):
    mu = jnp.mean(z, axis=0, keepdims=True)
    var = jnp.mean((z - mu) ** 2, axis=0, keepdims=True)
    return jnp.maximum((z - mu) / jnp.sqrt(var + _EPS) * g + b, 0.0)


def _prep_body(xu, xi, Wu, bu, gu, bbu, Wi, bi, gi, bbi, hu_o, hi_o):
    zu = jnp.dot(xu[...], Wu[...], preferred_element_type=jnp.float32) + bu[...]
    hu_o[...] = _bn_relu(zu, gu[...], bbu[...])
    zi = jnp.dot(xi[...], Wi[...], preferred_element_type=jnp.float32) + bi[...]
    hi_o[...] = _bn_relu(zi, gi[...], bbi[...])


def _layer_body(ci, P, cnt, h, Wl, bl, Wr, g, bb, out):
    n = h.shape[0]
    agg = (P[0, :n] + P[1, :n]) / jnp.maximum(cnt[ci, :n, :1], 1.0)
    z = (jnp.dot(agg, Wl[...], preferred_element_type=jnp.float32)
         + bl[...]
         + jnp.dot(h[...], Wr[...], preferred_element_type=jnp.float32))
    out[...] = _bn_relu(z, g[...], bb[...]) + h[...]


def _final_body(hu, hi, W, b, ou, oi):
    ou[...] = jnp.dot(hu[...], W[...], preferred_element_type=jnp.float32) + b[...]
    oi[...] = jnp.dot(hi[...], W[...], preferred_element_type=jnp.float32) + b[...]


def _r2(v):
    return v.reshape(1, -1)


def kernel(x_user, x_item, edge_ui, edge_iu, params):
    p = params
    N_U, H = x_user.shape
    N_I = x_item.shape[0]
    assert N_U == N_I
    E = edge_ui.shape[1]
    f32 = jnp.float32
    n_pad = _pad_dst(max(N_U, N_I))
    n_trash = n_pad - max(N_U, N_I)

    def trash_rows(k):
        return max(N_U, N_I) + (jnp.arange(k, dtype=jnp.int32) % n_trash)

    q = _C * _NW
    e_seg = ((E + q - 1) // q) * q
    per_w = e_seg // _NW
    n_chunk = per_w // _C

    def padded(row, fill):
        if e_seg == E:
            return row
        if fill is None:
            fill = jnp.zeros((e_seg - E,), jnp.int32)
        return jnp.concatenate([row, fill])

    tr_seg = trash_rows(e_seg - E) if e_seg > E else None
    src_ui = padded(edge_ui[0] + N_U, None).reshape(_NW, n_chunk, _C)
    dst_ui = padded(edge_ui[1], tr_seg).reshape(_NW, n_chunk, _C)
    src_iu = padded(edge_iu[0], None).reshape(_NW, n_chunk, _C)
    dst_iu = padded(edge_iu[1], tr_seg).reshape(_NW, n_chunk, _C)

    qc = _CC * _NS
    e_cnt = ((E + qc - 1) // qc) * qc
    nc_cnt = e_cnt // _NS // _CC

    def padded_c(row):
        return jnp.concatenate([row, trash_rows(e_cnt - E)])

    dst_cnt = jnp.concatenate([
        padded_c(edge_iu[1]).reshape(_NS, nc_cnt, _CC),
        padded_c(edge_ui[1]).reshape(_NS, nc_cnt, _CC)])

    prep = pl.pallas_call(
        _prep_body,
        out_shape=[jax.ShapeDtypeStruct((N_U, H), f32),
                   jax.ShapeDtypeStruct((N_I, H), f32)],
    )
    h_u, h_i = prep(x_user, x_item,
                    p['lin_user_W'], _r2(p['lin_user_b']),
                    _r2(p['in_bn_user_g']), _r2(p['in_bn_user_b']),
                    p['lin_item_W'], _r2(p['lin_item_b']),
                    _r2(p['in_bn_item_g']), _r2(p['in_bn_item_b']))

    cnt = _make_counts(n_pad, nc_cnt)(dst_cnt)

    seg = _make_seg_sum(n_pad, n_chunk, H)

    layer_i = pl.pallas_call(
        functools.partial(_layer_body, 1),
        out_shape=jax.ShapeDtypeStruct((N_I, H), f32),
    )
    layer_u = pl.pallas_call(
        functools.partial(_layer_body, 0),
        out_shape=jax.ShapeDtypeStruct((N_U, H), f32),
    )
    for l in range(3):
        h2n = jnp.concatenate([h_i, h_u])
        P_i = seg(h2n, src_ui, dst_ui)
        P_u = seg(h2n, src_iu, dst_iu)
        h_i_new = layer_i(P_i, cnt, h_i,
                          p[f'c{l}_ui_Wl'], _r2(p[f'c{l}_ui_bl']),
                          p[f'c{l}_ui_Wr'],
                          _r2(p[f'bn{l}_i_g']), _r2(p[f'bn{l}_i_b']))
        h_u_new = layer_u(P_u, cnt, h_u,
                          p[f'c{l}_iu_Wl'], _r2(p[f'c{l}_iu_bl']),
                          p[f'c{l}_iu_Wr'],
                          _r2(p[f'bn{l}_u_g']), _r2(p[f'bn{l}_u_b']))
        h_u, h_i = h_u_new, h_i_new

    final = pl.pallas_call(
        _final_body,
        out_shape=[jax.ShapeDtypeStruct((N_U, p['final_W'].shape[1]), f32),
                   jax.ShapeDtypeStruct((N_I, p['final_W'].shape[1]), f32)],
    )
    return final(h_u, h_i, p['final_W'], _r2(p['final_b']))


# speedup vs baseline: 1.9233x; 1.0019x over previous
import functools

import jax
import jax.numpy as jnp
from jax import lax
from jax.experimental import pallas as pl
from jax.experimental.pallas import tpu as pltpu
from jax.experimental.pallas import tpu_sc as plsc

_EPS = 1e-5

_NC = 2
_NS = 16
_NW = _NC * _NS
_C = 80
_CC = 128


def _pad_dst(n):
    q = _NS * 8
    return ((n + q - 1) // q) * q


def _chunks(total, step):
    out = []
    o = 0
    while o < total:
        s = min(step, total - o)
        out.append((o, s))
        o += s
    return out


def _seg_sum_body(n_pad, n_chunk, h, x_hbm, src_hbm, dst_hbm, out_hbm,
                  src_v, dst_v, rows_v, zero_v, acc_sh, sem):
    cid = lax.axis_index("c")
    sid = lax.axis_index("s")
    wid = cid * _NS + sid
    rows_per_sub = n_pad // _NS

    @pl.loop(0, 8)
    def _(i):
        for c in range(h // 16):
            zero_v[i, pl.ds(c * 16, 16)] = jnp.zeros((16,), jnp.float32)

    for off, sz in _chunks(rows_per_sub, 8):
        pltpu.sync_copy(zero_v.at[pl.ds(0, sz)],
                        acc_sh.at[pl.ds(sid * rows_per_sub + off, sz)])

    pltpu.sync_copy(src_hbm.at[wid], src_v)
    pltpu.sync_copy(dst_hbm.at[wid], dst_v)
    plsc.subcore_barrier()

    @pl.loop(0, n_chunk)
    def _(j):
        pltpu.async_copy(x_hbm.at[src_v.at[j]], rows_v, sem).wait()
        pltpu.sync_copy(rows_v, acc_sh.at[dst_v.at[j]], add=True)

    plsc.subcore_barrier()

    for off, sz in _chunks(rows_per_sub, 128):
        base = sid * rows_per_sub + off
        pltpu.sync_copy(acc_sh.at[pl.ds(base, sz)],
                        out_hbm.at[cid].at[pl.ds(base, sz)])


def _make_seg_sum(n_pad, n_chunk, h):
    mesh = plsc.VectorSubcoreMesh(core_axis_name="c", subcore_axis_name="s")
    return pl.kernel(
        functools.partial(_seg_sum_body, n_pad, n_chunk, h),
        out_type=jax.ShapeDtypeStruct((_NC, n_pad, h), jnp.float32),
        mesh=mesh,
        scratch_types=[
            pltpu.VMEM((n_chunk, _C), jnp.int32),
            pltpu.VMEM((n_chunk, _C), jnp.int32),
            pltpu.VMEM((_C, h), jnp.float32),
            pltpu.VMEM((8, h), jnp.float32),
            pltpu.VMEM_SHARED((n_pad, h), jnp.float32),
            pltpu.SemaphoreType.DMA,
        ],
    )


def _counts_body(n_pad, n_chunk, dst_hbm, cnt_hbm, dst_v, ones_v, acc_sh):
    cid = lax.axis_index("c")
    sid = lax.axis_index("s")
    wid = cid * _NS + sid
    rows_per_sub = n_pad // _NS

    @pl.loop(0, _CC)
    def _(i):
        for c in range(8):
            ones_v[i, pl.ds(c * 16, 16)] = jnp.zeros((16,), jnp.float32)

    for off, sz in _chunks(rows_per_sub, _CC):
        pltpu.sync_copy(ones_v.at[pl.ds(0, sz)],
                        acc_sh.at[pl.ds(sid * rows_per_sub + off, sz)])

    @pl.loop(0, _CC)
    def _(i):
        for c in range(8):
            ones_v[i, pl.ds(c * 16, 16)] = jnp.ones((16,), jnp.float32)

    pltpu.sync_copy(dst_hbm.at[wid], dst_v)
    plsc.subcore_barrier()

    @pl.loop(0, n_chunk)
    def _(j):
        pltpu.sync_copy(ones_v, acc_sh.at[dst_v.at[j]], add=True)

    plsc.subcore_barrier()

    for off, sz in _chunks(rows_per_sub, 128):
        base = sid * rows_per_sub + off
        pltpu.sync_copy(acc_sh.at[pl.ds(base, sz)],
                        cnt_hbm.at[cid].at[pl.ds(base, sz)])


def _make_counts(n_pad, n_chunk):
    mesh = plsc.VectorSubcoreMesh(core_axis_name="c", subcore_axis_name="s")
    return pl.kernel(
        functools.partial(_counts_body, n_pad, n_chunk),
        out_type=jax.ShapeDtypeStruct((_NC, n_pad, 128), jnp.float32),
        mesh=mesh,
        scratch_types=[
            pltpu.VMEM((n_chunk, _CC), jnp.int32),
            pltpu.VMEM((_CC, 128), jnp.float32),
            pltpu.VMEM_SHARED((n_pad, 128), jnp.float32),
        ],
    )


def _bn_relu(z, g, b):
    mu = jnp.mean(z, axis=0, keepdims=True)
    var = jnp.mean((z - mu) ** 2, axis=0, keepdims=True)
    return jnp.maximum((z - mu) / jnp.sqrt(var + _EPS) * g + b, 0.0)


def _prep_body(xu, xi, Wu, bu, gu, bbu, Wi, bi, gi, bbi, hu_o, hi_o):
    zu = jnp.dot(xu[...], Wu[...], preferred_element_type=jnp.float32) + bu[...]
    hu_o[...] = _bn_relu(zu, gu[...], bbu[...])
    zi = jnp.dot(xi[...], Wi[...], preferred_element_type=jnp.float32) + bi[...]
    hi_o[...] = _bn_relu(zi, gi[...], bbi[...])


def _layer_body(P, cnt, h, Wl, bl, Wr, g, bb, out):
    n = h.shape[0]
    agg = (P[0, :n] + P[1, :n]) / jnp.maximum(cnt[:n, :1], 1.0)
    z = (jnp.dot(agg, Wl[...], preferred_element_type=jnp.float32)
         + bl[...]
         + jnp.dot(h[...], Wr[...], preferred_element_type=jnp.float32))
    out[...] = _bn_relu(z, g[...], bb[...]) + h[...]


def _final_body(hu, hi, W, b, ou, oi):
    ou[...] = jnp.dot(hu[...], W[...], preferred_element_type=jnp.float32) + b[...]
    oi[...] = jnp.dot(hi[...], W[...], preferred_element_type=jnp.float32) + b[...]


def _r2(v):
    return v.reshape(1, -1)


def kernel(x_user, x_item, edge_ui, edge_iu, params):
    p = params
    N_U, H = x_user.shape
    N_I = x_item.shape[0]
    assert N_U == N_I
    E = edge_ui.shape[1]
    f32 = jnp.float32
    n_pad = _pad_dst(max(N_U, N_I))
    n_trash = n_pad - max(N_U, N_I)

    def trash_rows(k):
        return max(N_U, N_I) + (jnp.arange(k, dtype=jnp.int32) % n_trash)

    q = _C * _NW
    e_seg = ((E + q - 1) // q) * q
    per_w = e_seg // _NW
    n_chunk = per_w // _C

    def padded(row, fill):
        if e_seg == E:
            return row
        if fill is None:
            fill = jnp.zeros((e_seg - E,), jnp.int32)
        return jnp.concatenate([row, fill])

    tr_seg = trash_rows(e_seg - E) if e_seg > E else None
    src_ui = padded(edge_ui[0] + N_U, None).reshape(_NW, n_chunk, _C)
    dst_ui = padded(edge_ui[1], tr_seg).reshape(_NW, n_chunk, _C)
    src_iu = padded(edge_iu[0], None).reshape(_NW, n_chunk, _C)
    dst_iu = padded(edge_iu[1], tr_seg).reshape(_NW, n_chunk, _C)

    qc = _CC * _NS
    e_cnt = ((E + qc - 1) // qc) * qc
    nc_cnt = e_cnt // _NS // _CC

    def padded_c(row):
        return jnp.concatenate([row, trash_rows(e_cnt - E)])

    dst_cnt = jnp.concatenate([
        padded_c(edge_iu[1]).reshape(_NS, nc_cnt, _CC),
        padded_c(edge_ui[1]).reshape(_NS, nc_cnt, _CC)])

    prep = pl.pallas_call(
        _prep_body,
        out_shape=[jax.ShapeDtypeStruct((N_U, H), f32),
                   jax.ShapeDtypeStruct((N_I, H), f32)],
    )
    h_u, h_i = prep(x_user, x_item,
                    p['lin_user_W'], _r2(p['lin_user_b']),
                    _r2(p['in_bn_user_g']), _r2(p['in_bn_user_b']),
                    p['lin_item_W'], _r2(p['lin_item_b']),
                    _r2(p['in_bn_item_g']), _r2(p['in_bn_item_b']))

    cnt = _make_counts(n_pad, nc_cnt)(dst_cnt)
    cnt_u, cnt_i = cnt[0], cnt[1]

    seg = _make_seg_sum(n_pad, n_chunk, H)

    layer_i = pl.pallas_call(
        _layer_body,
        out_shape=jax.ShapeDtypeStruct((N_I, H), f32),
    )
    layer_u = pl.pallas_call(
        _layer_body,
        out_shape=jax.ShapeDtypeStruct((N_U, H), f32),
    )
    for l in range(3):
        h2n = jnp.concatenate([h_i, h_u])
        P_i = seg(h2n, src_ui, dst_ui)
        P_u = seg(h2n, src_iu, dst_iu)
        h_i_new = layer_i(P_i, cnt_i, h_i,
                          p[f'c{l}_ui_Wl'], _r2(p[f'c{l}_ui_bl']),
                          p[f'c{l}_ui_Wr'],
                          _r2(p[f'bn{l}_i_g']), _r2(p[f'bn{l}_i_b']))
        h_u_new = layer_u(P_u, cnt_u, h_u,
                          p[f'c{l}_iu_Wl'], _r2(p[f'c{l}_iu_bl']),
                          p[f'c{l}_iu_Wr'],
                          _r2(p[f'bn{l}_u_g']), _r2(p[f'bn{l}_u_b']))
        h_u, h_i = h_u_new, h_i_new

    final = pl.pallas_call(
        _final_body,
        out_shape=[jax.ShapeDtypeStruct((N_U, p['final_W'].shape[1]), f32),
                   jax.ShapeDtypeStruct((N_I, p['final_W'].shape[1]), f32)],
    )
    return final(h_u, h_i, p['final_W'], _r2(p['final_b']))


# speedup vs baseline: 1.9725x; 1.0256x over previous
import functools

import jax
import jax.numpy as jnp
from jax import lax
from jax.experimental import pallas as pl
from jax.experimental.pallas import tpu as pltpu
from jax.experimental.pallas import tpu_sc as plsc

_EPS = 1e-5

_NC = 2
_NS = 16
_NW = _NC * _NS
_C = 80
_CC = 128


def _pad_dst(n):
    q = _NS * 8
    return ((n + q - 1) // q) * q


def _chunks(total, step):
    out = []
    o = 0
    while o < total:
        s = min(step, total - o)
        out.append((o, s))
        o += s
    return out


def _seg_sum_body(n_pad, n_chunk, h, x_hbm, src_hbm, dst_hbm, out_hbm,
                  src_v, dst_v, rows_v, zero_v, acc_sh, sem):
    cid = lax.axis_index("c")
    sid = lax.axis_index("s")
    wid = cid * _NS + sid
    rows_per_sub = n_pad // _NS

    @pl.loop(0, 8)
    def _(i):
        for c in range(h // 16):
            zero_v[i, pl.ds(c * 16, 16)] = jnp.zeros((16,), jnp.float32)

    for off, sz in _chunks(rows_per_sub, 8):
        pltpu.sync_copy(zero_v.at[pl.ds(0, sz)],
                        acc_sh.at[pl.ds(sid * rows_per_sub + off, sz)])

    pltpu.sync_copy(src_hbm.at[wid], src_v)
    pltpu.sync_copy(dst_hbm.at[wid], dst_v)
    plsc.subcore_barrier()

    @pl.loop(0, n_chunk)
    def _(j):
        pltpu.async_copy(x_hbm.at[src_v.at[j]], rows_v, sem).wait()
        pltpu.sync_copy(rows_v, acc_sh.at[dst_v.at[j]], add=True)

    plsc.subcore_barrier()

    for off, sz in _chunks(rows_per_sub, 128):
        base = sid * rows_per_sub + off
        pltpu.sync_copy(acc_sh.at[pl.ds(base, sz)],
                        out_hbm.at[cid].at[pl.ds(base, sz)])


def _make_seg_sum(n_pad, n_chunk, h):
    mesh = plsc.VectorSubcoreMesh(core_axis_name="c", subcore_axis_name="s")
    return pl.kernel(
        functools.partial(_seg_sum_body, n_pad, n_chunk, h),
        out_type=jax.ShapeDtypeStruct((_NC, n_pad, h), jnp.float32),
        mesh=mesh,
        scratch_types=[
            pltpu.VMEM((n_chunk, _C), jnp.int32),
            pltpu.VMEM((n_chunk, _C), jnp.int32),
            pltpu.VMEM((_C, h), jnp.float32),
            pltpu.VMEM((8, h), jnp.float32),
            pltpu.VMEM_SHARED((n_pad, h), jnp.float32),
            pltpu.SemaphoreType.DMA,
        ],
    )


def _counts_body(n_pad, n_chunk, dst_hbm, cnt_hbm, dst_v, ones_v, acc_sh):
    cid = lax.axis_index("c")
    sid = lax.axis_index("s")
    wid = cid * _NS + sid
    rows_per_sub = n_pad // _NS

    @pl.loop(0, _CC)
    def _(i):
        for c in range(8):
            ones_v[i, pl.ds(c * 16, 16)] = jnp.zeros((16,), jnp.float32)

    for off, sz in _chunks(rows_per_sub, _CC):
        pltpu.sync_copy(ones_v.at[pl.ds(0, sz)],
                        acc_sh.at[pl.ds(sid * rows_per_sub + off, sz)])

    @pl.loop(0, _CC)
    def _(i):
        for c in range(8):
            ones_v[i, pl.ds(c * 16, 16)] = jnp.ones((16,), jnp.float32)

    pltpu.sync_copy(dst_hbm.at[wid], dst_v)
    plsc.subcore_barrier()

    @pl.loop(0, n_chunk)
    def _(j):
        pltpu.sync_copy(ones_v, acc_sh.at[dst_v.at[j]], add=True)

    plsc.subcore_barrier()

    for off, sz in _chunks(rows_per_sub, 128):
        base = sid * rows_per_sub + off
        pltpu.sync_copy(acc_sh.at[pl.ds(base, sz)],
                        cnt_hbm.at[cid].at[pl.ds(base, sz)])


def _make_counts(n_pad, n_chunk):
    mesh = plsc.VectorSubcoreMesh(core_axis_name="c", subcore_axis_name="s")
    return pl.kernel(
        functools.partial(_counts_body, n_pad, n_chunk),
        out_type=jax.ShapeDtypeStruct((_NC, n_pad, 128), jnp.float32),
        mesh=mesh,
        scratch_types=[
            pltpu.VMEM((n_chunk, _CC), jnp.int32),
            pltpu.VMEM((_CC, 128), jnp.float32),
            pltpu.VMEM_SHARED((n_pad, 128), jnp.float32),
        ],
    )


def _bn_relu(z, g, b):
    mu = jnp.mean(z, axis=0, keepdims=True)
    var = jnp.mean((z - mu) ** 2, axis=0, keepdims=True)
    return jnp.maximum((z - mu) / jnp.sqrt(var + _EPS) * g + b, 0.0)


def _prep_body(xu, xi, Wu, bu, gu, bbu, Wi, bi, gi, bbi, hu_o, hi_o):
    zu = jnp.dot(xu[...], Wu[...], preferred_element_type=jnp.float32) + bu[...]
    hu_o[...] = _bn_relu(zu, gu[...], bbu[...])
    zi = jnp.dot(xi[...], Wi[...], preferred_element_type=jnp.float32) + bi[...]
    hi_o[...] = _bn_relu(zi, gi[...], bbi[...])


def _layer_body(P, cnt, h, Wl, bl, Wr, g, bb, out):
    n = h.shape[0]
    agg = (P[0, :n] + P[1, :n]) / jnp.maximum(cnt[:n, :1], 1.0)
    z = (jnp.dot(agg, Wl[...], preferred_element_type=jnp.float32)
         + bl[...]
         + jnp.dot(h[...], Wr[...], preferred_element_type=jnp.float32))
    out[...] = _bn_relu(z, g[...], bb[...]) + h[...]


def _final_body(hu, hi, W, b, ou, oi):
    ou[...] = jnp.dot(hu[...], W[...], preferred_element_type=jnp.float32) + b[...]
    oi[...] = jnp.dot(hi[...], W[...], preferred_element_type=jnp.float32) + b[...]


def _r2(v):
    return v.reshape(1, -1)


def kernel(x_user, x_item, edge_ui, edge_iu, params):
    p = params
    N_U, H = x_user.shape
    N_I = x_item.shape[0]
    assert N_U == N_I
    E = edge_ui.shape[1]
    f32 = jnp.float32
    n_pad = _pad_dst(max(N_U, N_I))
    n_trash = n_pad - max(N_U, N_I)

    def trash_rows(k):
        return max(N_U, N_I) + (jnp.arange(k, dtype=jnp.int32) % n_trash)

    q = _C * _NW
    e_seg = ((E + q - 1) // q) * q
    per_w = e_seg // _NW
    n_chunk = per_w // _C

    def padded(row, fill):
        if e_seg == E:
            return row
        if fill is None:
            fill = jnp.zeros((e_seg - E,), jnp.int32)
        return jnp.concatenate([row, fill])

    tr_seg = trash_rows(e_seg - E) if e_seg > E else None
    src_ui = padded(edge_ui[0], None).reshape(_NW, n_chunk, _C)
    dst_ui = padded(edge_ui[1], tr_seg).reshape(_NW, n_chunk, _C)
    src_iu = padded(edge_iu[0], None).reshape(_NW, n_chunk, _C)
    dst_iu = padded(edge_iu[1], tr_seg).reshape(_NW, n_chunk, _C)

    qc = _CC * _NS
    e_cnt = ((E + qc - 1) // qc) * qc
    nc_cnt = e_cnt // _NS // _CC

    def padded_c(row):
        return jnp.concatenate([row, trash_rows(e_cnt - E)])

    dst_cnt = jnp.concatenate([
        padded_c(edge_iu[1]).reshape(_NS, nc_cnt, _CC),
        padded_c(edge_ui[1]).reshape(_NS, nc_cnt, _CC)])

    prep = pl.pallas_call(
        _prep_body,
        out_shape=[jax.ShapeDtypeStruct((N_U, H), f32),
                   jax.ShapeDtypeStruct((N_I, H), f32)],
    )
    h_u, h_i = prep(x_user, x_item,
                    p['lin_user_W'], _r2(p['lin_user_b']),
                    _r2(p['in_bn_user_g']), _r2(p['in_bn_user_b']),
                    p['lin_item_W'], _r2(p['lin_item_b']),
                    _r2(p['in_bn_item_g']), _r2(p['in_bn_item_b']))

    cnt = _make_counts(n_pad, nc_cnt)(dst_cnt)
    cnt_u, cnt_i = cnt[0], cnt[1]

    seg = _make_seg_sum(n_pad, n_chunk, H)

    layer_i = pl.pallas_call(
        _layer_body,
        out_shape=jax.ShapeDtypeStruct((N_I, H), f32),
    )
    layer_u = pl.pallas_call(
        _layer_body,
        out_shape=jax.ShapeDtypeStruct((N_U, H), f32),
    )
    for l in range(3):
        P_i = seg(h_u, src_ui, dst_ui)
        P_u = seg(h_i, src_iu, dst_iu)
        h_i_new = layer_i(P_i, cnt_i, h_i,
                          p[f'c{l}_ui_Wl'], _r2(p[f'c{l}_ui_bl']),
                          p[f'c{l}_ui_Wr'],
                          _r2(p[f'bn{l}_i_g']), _r2(p[f'bn{l}_i_b']))
        h_u_new = layer_u(P_u, cnt_u, h_u,
                          p[f'c{l}_iu_Wl'], _r2(p[f'c{l}_iu_bl']),
                          p[f'c{l}_iu_Wr'],
                          _r2(p[f'bn{l}_u_g']), _r2(p[f'bn{l}_u_b']))
        h_u, h_i = h_u_new, h_i_new

    final = pl.pallas_call(
        _final_body,
        out_shape=[jax.ShapeDtypeStruct((N_U, p['final_W'].shape[1]), f32),
                   jax.ShapeDtypeStruct((N_I, p['final_W'].shape[1]), f32)],
    )
    return final(h_u, h_i, p['final_W'], _r2(p['final_b']))
